# baseline (device time: 61124 ns/iter reference)
import jax
import jax.numpy as jnp
from jax import lax
from jax.experimental import pallas as pl
from jax.experimental.pallas import tpu as pltpu

N_DEV = 4
M_PER = 1024
K = 4096
N_PER = 512
XC = M_PER // 4
MH = M_PER // 2

D_ORDER = (2, 1, 3, 4)


def kernel(x, w_mat):
    def body(x_hbm, w_hbm, out_ref, xs, xb, ws, wb, yb, rb,
             xsems, wsem, send_sems, recv_sems):
        me = lax.axis_index("i")

        def x_copy(c):
            return pltpu.make_async_copy(
                x_hbm.at[pl.ds(c * XC, XC)], xs.at[c % 2], xsems.at[c % 2])

        def w_copy(d):
            peer = (me + d) % N_DEV
            return pltpu.make_async_copy(
                w_hbm.at[:, pl.ds(peer * N_PER, N_PER)], ws, wsem)

        x_copy(0).start()
        w_copy(D_ORDER[0]).start()

        barrier_sem = pltpu.get_barrier_semaphore()
        for d in (1, 2, 3):
            pl.semaphore_signal(
                barrier_sem, inc=1,
                device_id=((me + d) % N_DEV,),
                device_id_type=pl.DeviceIdType.MESH,
            )
        pl.semaphore_wait(barrier_sem, 3)

        for c in range(4):
            if c + 1 < 4:
                x_copy(c + 1).start()
            x_copy(c).wait()
            xb[c * XC:(c + 1) * XC, :] = xs[c % 2].astype(jnp.bfloat16)

        w_copy(D_ORDER[0]).wait()
        wb[0] = ws[:, :].astype(jnp.bfloat16)
        w_copy(D_ORDER[1]).start()

        sends = []
        for i, d in enumerate(D_ORDER):
            cur = i % 2
            if i + 1 < 4:
                w_copy(D_ORDER[i + 1]).wait()
                wb[1 - cur] = ws[:, :].astype(jnp.bfloat16)
            if i + 2 < 4:
                w_copy(D_ORDER[i + 2]).start()
            for h in (0, 1):
                yh = jnp.dot(xb[h * MH:(h + 1) * MH, :], wb[cur],
                             preferred_element_type=jnp.float32)
                yh = yh * jax.nn.sigmoid(yh)
                if d < 4:
                    yb[d - 1, h * MH:(h + 1) * MH, :] = yh.astype(jnp.bfloat16)
                    rdma = pltpu.make_async_remote_copy(
                        src_ref=yb.at[d - 1, pl.ds(h * MH, MH)],
                        dst_ref=rb.at[3 - d, pl.ds(h * MH, MH)],
                        send_sem=send_sems.at[d - 1, h],
                        recv_sem=recv_sems.at[3 - d, h],
                        device_id=((me + d) % N_DEV,),
                        device_id_type=pl.DeviceIdType.MESH,
                    )
                    rdma.start()
                    sends.append(rdma)
                else:
                    out_ref[pl.ds(me * M_PER + h * MH, MH), :] = yh

        for s in (2, 3, 1):
            for h in (0, 1):
                recv = pltpu.make_async_remote_copy(
                    src_ref=yb.at[0, pl.ds(h * MH, MH)],
                    dst_ref=rb.at[s - 1, pl.ds(h * MH, MH)],
                    send_sem=send_sems.at[0, h],
                    recv_sem=recv_sems.at[s - 1, h],
                    device_id=(me,),
                    device_id_type=pl.DeviceIdType.MESH,
                )
                recv.wait_recv()
            origin = (me + s) % N_DEV
            out_ref[pl.ds(origin * M_PER, M_PER), :] = (
                rb[s - 1].astype(jnp.float32))

        for rdma in sends:
            rdma.wait_send()

    out_shape = jax.ShapeDtypeStruct((N_DEV * M_PER, N_PER), jnp.float32)
    return pl.pallas_call(
        body,
        out_shape=out_shape,
        in_specs=[
            pl.BlockSpec(memory_space=pltpu.MemorySpace.HBM),
            pl.BlockSpec(memory_space=pltpu.MemorySpace.HBM),
        ],
        out_specs=pl.BlockSpec(memory_space=pltpu.VMEM),
        scratch_shapes=[
            pltpu.VMEM((2, XC, K), jnp.float32),
            pltpu.VMEM((M_PER, K), jnp.bfloat16),
            pltpu.VMEM((K, N_PER), jnp.float32),
            pltpu.VMEM((2, K, N_PER), jnp.bfloat16),
            pltpu.VMEM((3, M_PER, N_PER), jnp.bfloat16),
            pltpu.VMEM((3, M_PER, N_PER), jnp.bfloat16),
            pltpu.SemaphoreType.DMA((2,)),
            pltpu.SemaphoreType.DMA,
            pltpu.SemaphoreType.DMA((3, 2)),
            pltpu.SemaphoreType.DMA((3, 2)),
        ],
        compiler_params=pltpu.CompilerParams(
            collective_id=0,
            vmem_limit_bytes=62 * 1024 * 1024,
        ),
    )(x, w_mat)


# device time: 50797 ns/iter; 1.2033x vs baseline; 1.2033x over previous
import jax
import jax.numpy as jnp
from jax import lax
from jax.experimental import pallas as pl
from jax.experimental.pallas import tpu as pltpu

N_DEV = 4
M_PER = 1024
K = 4096
N_PER = 512
XC = M_PER // 4
MH = M_PER // 2

D_ORDER = (2, 1, 3, 4)


def kernel(x, w_mat):
    def body(x_hbm, w_hbm, out_ref, xs, xb, ws, yb, rb,
             xsems, wsems, send_sems, recv_sems):
        me = lax.axis_index("i")

        def x_copy(c):
            return pltpu.make_async_copy(
                x_hbm.at[pl.ds(c * XC, XC)], xs.at[c % 2], xsems.at[c % 2])

        def w_copy(i):
            peer = (me + D_ORDER[i]) % N_DEV
            return pltpu.make_async_copy(
                w_hbm.at[:, pl.ds(peer * N_PER, N_PER)],
                ws.at[i % 2], wsems.at[i % 2])

        x_copy(0).start()
        w_copy(0).start()

        barrier_sem = pltpu.get_barrier_semaphore()
        for d in (1, 2, 3):
            pl.semaphore_signal(
                barrier_sem, inc=1,
                device_id=((me + d) % N_DEV,),
                device_id_type=pl.DeviceIdType.MESH,
            )
        pl.semaphore_wait(barrier_sem, 3)

        for c in range(4):
            if c + 1 < 4:
                x_copy(c + 1).start()
            x_copy(c).wait()
            xb[c * XC:(c + 1) * XC, :] = xs[c % 2].astype(jnp.bfloat16)

        sends = []
        for i, d in enumerate(D_ORDER):
            if i + 1 < 4:
                w_copy(i + 1).start()
            w_copy(i).wait()
            wv = ws[i % 2].astype(jnp.bfloat16)
            for h in (0, 1):
                yh = jnp.dot(xb[h * MH:(h + 1) * MH, :], wv,
                             preferred_element_type=jnp.float32)
                yh = yh * jax.nn.sigmoid(yh)
                if d < 4:
                    yb[d - 1, h * MH:(h + 1) * MH, :] = yh.astype(jnp.bfloat16)
                    rdma = pltpu.make_async_remote_copy(
                        src_ref=yb.at[d - 1, pl.ds(h * MH, MH)],
                        dst_ref=rb.at[3 - d, pl.ds(h * MH, MH)],
                        send_sem=send_sems.at[d - 1, h],
                        recv_sem=recv_sems.at[3 - d, h],
                        device_id=((me + d) % N_DEV,),
                        device_id_type=pl.DeviceIdType.MESH,
                    )
                    rdma.start()
                    sends.append(rdma)
                else:
                    out_ref[pl.ds(me * M_PER + h * MH, MH), :] = yh

        for s in (2, 3, 1):
            for h in (0, 1):
                recv = pltpu.make_async_remote_copy(
                    src_ref=yb.at[0, pl.ds(h * MH, MH)],
                    dst_ref=rb.at[s - 1, pl.ds(h * MH, MH)],
                    send_sem=send_sems.at[0, h],
                    recv_sem=recv_sems.at[s - 1, h],
                    device_id=(me,),
                    device_id_type=pl.DeviceIdType.MESH,
                )
                recv.wait_recv()
            origin = (me + s) % N_DEV
            out_ref[pl.ds(origin * M_PER, M_PER), :] = (
                rb[s - 1].astype(jnp.float32))

        for rdma in sends:
            rdma.wait_send()

    out_shape = jax.ShapeDtypeStruct((N_DEV * M_PER, N_PER), jnp.float32)
    return pl.pallas_call(
        body,
        out_shape=out_shape,
        in_specs=[
            pl.BlockSpec(memory_space=pltpu.MemorySpace.HBM),
            pl.BlockSpec(memory_space=pltpu.MemorySpace.HBM),
        ],
        out_specs=pl.BlockSpec(memory_space=pltpu.VMEM),
        scratch_shapes=[
            pltpu.VMEM((2, XC, K), jnp.float32),
            pltpu.VMEM((M_PER, K), jnp.bfloat16),
            pltpu.VMEM((2, K, N_PER), jnp.float32),
            pltpu.VMEM((3, M_PER, N_PER), jnp.bfloat16),
            pltpu.VMEM((3, M_PER, N_PER), jnp.bfloat16),
            pltpu.SemaphoreType.DMA((2,)),
            pltpu.SemaphoreType.DMA((2,)),
            pltpu.SemaphoreType.DMA((3, 2)),
            pltpu.SemaphoreType.DMA((3, 2)),
        ],
        compiler_params=pltpu.CompilerParams(
            collective_id=0,
            vmem_limit_bytes=62 * 1024 * 1024,
        ),
    )(x, w_mat)


# device time: 47587 ns/iter; 1.2845x vs baseline; 1.0675x over previous
import jax
import jax.numpy as jnp
from jax import lax
from jax.experimental import pallas as pl
from jax.experimental.pallas import tpu as pltpu

N_DEV = 4
M_PER = 1024
K = 4096
N_PER = 512
XC = M_PER // 4

D_ORDER = (2, 1, 3, 4)


def kernel(x, w_mat):
    def body(x_hbm, w_hbm, out_ref, xs, xb, ws, yb, rb,
             xsems, wsems, send_sems, recv_sems):
        me = lax.axis_index("i")

        def x_copy(c):
            return pltpu.make_async_copy(
                x_hbm.at[pl.ds(c * XC, XC)], xs.at[c % 2], xsems.at[c % 2])

        def w_copy(i):
            peer = (me + D_ORDER[i]) % N_DEV
            return pltpu.make_async_copy(
                w_hbm.at[:, pl.ds(peer * N_PER, N_PER)],
                ws.at[i % 2], wsems.at[i % 2])

        sends = []

        def quarter(d, q, yq):
            yq = yq * jax.nn.sigmoid(yq)
            if d < 4:
                yb[d - 1, q * XC:(q + 1) * XC, :] = yq.astype(jnp.bfloat16)
                rdma = pltpu.make_async_remote_copy(
                    src_ref=yb.at[d - 1, pl.ds(q * XC, XC)],
                    dst_ref=rb.at[3 - d, pl.ds(q * XC, XC)],
                    send_sem=send_sems.at[d - 1, q],
                    recv_sem=recv_sems.at[3 - d, q],
                    device_id=((me + d) % N_DEV,),
                    device_id_type=pl.DeviceIdType.MESH,
                )
                rdma.start()
                sends.append(rdma)
            else:
                out_ref[pl.ds(me * M_PER + q * XC, XC), :] = yq

        x_copy(0).start()
        w_copy(0).start()

        barrier_sem = pltpu.get_barrier_semaphore()
        for d in (1, 2, 3):
            pl.semaphore_signal(
                barrier_sem, inc=1,
                device_id=((me + d) % N_DEV,),
                device_id_type=pl.DeviceIdType.MESH,
            )
        pl.semaphore_wait(barrier_sem, 3)

        w_copy(0).wait()
        w_copy(1).start()
        wv = ws[0].astype(jnp.bfloat16)
        for c in range(4):
            if c + 1 < 4:
                x_copy(c + 1).start()
            x_copy(c).wait()
            xv = xs[c % 2].astype(jnp.bfloat16)
            xb[c * XC:(c + 1) * XC, :] = xv
            yq = jnp.dot(xv, wv, preferred_element_type=jnp.float32)
            quarter(D_ORDER[0], c, yq)

        for i in (1, 2, 3):
            if i + 1 < 4:
                w_copy(i + 1).start()
            w_copy(i).wait()
            wv = ws[i % 2].astype(jnp.bfloat16)
            for q in range(4):
                yq = jnp.dot(xb[q * XC:(q + 1) * XC, :], wv,
                             preferred_element_type=jnp.float32)
                quarter(D_ORDER[i], q, yq)

        for s in (2, 3, 1):
            for q in range(4):
                recv = pltpu.make_async_remote_copy(
                    src_ref=yb.at[0, pl.ds(q * XC, XC)],
                    dst_ref=rb.at[s - 1, pl.ds(q * XC, XC)],
                    send_sem=send_sems.at[0, q],
                    recv_sem=recv_sems.at[s - 1, q],
                    device_id=(me,),
                    device_id_type=pl.DeviceIdType.MESH,
                )
                recv.wait_recv()
            origin = (me + s) % N_DEV
            out_ref[pl.ds(origin * M_PER, M_PER), :] = (
                rb[s - 1].astype(jnp.float32))

        for rdma in sends:
            rdma.wait_send()

    out_shape = jax.ShapeDtypeStruct((N_DEV * M_PER, N_PER), jnp.float32)
    return pl.pallas_call(
        body,
        out_shape=out_shape,
        in_specs=[
            pl.BlockSpec(memory_space=pltpu.MemorySpace.HBM),
            pl.BlockSpec(memory_space=pltpu.MemorySpace.HBM),
        ],
        out_specs=pl.BlockSpec(memory_space=pltpu.VMEM),
        scratch_shapes=[
            pltpu.VMEM((2, XC, K), jnp.float32),
            pltpu.VMEM((M_PER, K), jnp.bfloat16),
            pltpu.VMEM((2, K, N_PER), jnp.float32),
            pltpu.VMEM((3, M_PER, N_PER), jnp.bfloat16),
            pltpu.VMEM((3, M_PER, N_PER), jnp.bfloat16),
            pltpu.SemaphoreType.DMA((2,)),
            pltpu.SemaphoreType.DMA((2,)),
            pltpu.SemaphoreType.DMA((3, 4)),
            pltpu.SemaphoreType.DMA((3, 4)),
        ],
        compiler_params=pltpu.CompilerParams(
            collective_id=0,
            vmem_limit_bytes=62 * 1024 * 1024,
        ),
    )(x, w_mat)
